# baseline (speedup 1.0000x reference)
import numpy as np
import jax
import jax.numpy as jnp
from jax.experimental import pallas as pl
from jax.experimental.pallas import tpu as pltpu

N = 10000
E = 160000
DIN = 128
DQ = 32
DK = 16
NB = 32
HID = 16
M0, M1, M2 = 16, 8, 4
DV = M0 + M1 + M2
DOUT = M0 + 3 * M1 + 5 * M2

EB = 2000
NSTEPS = E // EB
QB = 2000

_R1 = np.kron(np.eye(M1), np.ones((1, 3))).astype(np.float32)
_T1 = np.kron(np.ones((1, M1)), np.eye(3)).astype(np.float32)
_R2 = np.kron(np.eye(M2), np.ones((1, 5))).astype(np.float32)
_T2 = np.kron(np.ones((1, M2)), np.eye(5)).astype(np.float32)


def _q_kernel(x_ref, wq_ref, q_ref):
    q_ref[...] = jnp.dot(x_ref[...], wq_ref[...],
                         preferred_element_type=jnp.float32) / (DIN ** 0.5)


def _edge_kernel(vec_ref, xs_ref, qd_ref, wkmat_ref, wvmat_ref, kw1_ref,
                 vw1_ref, wtpq_ref, wsim_ref, r1_ref, t1_ref, r2_ref, t2_ref,
                 logits_ref, v_ref):
    vec = vec_ref[...]
    d2 = jnp.sum(vec * vec, axis=1, keepdims=True)
    d = jnp.sqrt(d2)
    u = vec / (d + 1e-9)
    sh1 = (3.0 ** 0.5) * u
    ux = u[:, 0:1]
    uy = u[:, 1:2]
    uz = u[:, 2:3]
    c2 = 15.0 ** 0.5
    sh2 = jnp.concatenate([
        c2 * ux * uy,
        c2 * uy * uz,
        (5.0 ** 0.5) * 0.5 * (3.0 * uz * uz - 1.0),
        c2 * ux * uz,
        0.5 * c2 * (ux * ux - uy * uy),
    ], axis=1)

    centers = jax.lax.broadcasted_iota(jnp.int32, (1, NB), 1).astype(jnp.float32) * (3.5 / (NB - 1))
    width = 3.5 / NB
    b = jnp.exp(-0.5 * ((d - centers) / width) ** 2)
    bn = b * (NB ** 0.5) / (jnp.sqrt(jnp.sum(b * b, axis=1, keepdims=True)) + 1e-9)

    hk = jax.nn.silu(jnp.dot(bn, kw1_ref[...],
                             preferred_element_type=jnp.float32) / (NB ** 0.5))
    hv = jax.nn.silu(jnp.dot(bn, vw1_ref[...],
                             preferred_element_type=jnp.float32) / (NB ** 0.5))

    xs = xs_ref[...]
    qd = qd_ref[...]

    tk = jnp.dot(xs, wkmat_ref[...], preferred_element_type=jnp.float32)
    k = jnp.zeros((EB, DK), dtype=jnp.float32)
    for h in range(HID):
        k = k + hk[:, h:h + 1] * tk[:, h * DK:(h + 1) * DK]
    k = k / (DIN ** 0.5) + jnp.dot(qd, wtpq_ref[...],
                                   preferred_element_type=jnp.float32) / (DQ ** 0.5)

    a = jnp.dot(qd, wsim_ref[...], preferred_element_type=jnp.float32)
    logits = jnp.sum(a * k, axis=1, keepdims=True) / ((DQ * DK) ** 0.5)

    tv = jnp.dot(xs, wvmat_ref[...], preferred_element_type=jnp.float32)
    s = jnp.zeros((EB, DV), dtype=jnp.float32)
    for h in range(HID):
        s = s + hv[:, h:h + 1] * tv[:, h * DV:(h + 1) * DV]
    s = s / (DIN ** 0.5)

    s0 = s[:, :M0]
    s1 = s[:, M0:M0 + M1]
    s2 = s[:, M0 + M1:]
    v1 = jnp.dot(s1, r1_ref[...], preferred_element_type=jnp.float32) * \
        jnp.dot(sh1, t1_ref[...], preferred_element_type=jnp.float32)
    v2 = jnp.dot(s2, r2_ref[...], preferred_element_type=jnp.float32) * \
        jnp.dot(sh2, t2_ref[...], preferred_element_type=jnp.float32)

    logits_ref[...] = logits
    v_ref[...] = jnp.concatenate([s0, v1, v2], axis=1)


def kernel(x, pos, edge_index, W_query, W_tpq, W_sim, kW1, kW2, vW1, vW2):
    src = edge_index[0]
    dst = edge_index[1]

    q = pl.pallas_call(
        _q_kernel,
        grid=(N // QB,),
        in_specs=[
            pl.BlockSpec((QB, DIN), lambda i: (i, 0)),
            pl.BlockSpec((DIN, DQ), lambda i: (0, 0)),
        ],
        out_specs=pl.BlockSpec((QB, DQ), lambda i: (i, 0)),
        out_shape=jax.ShapeDtypeStruct((N, DQ), jnp.float32),
    )(x, W_query)

    xs = jnp.take(x, src, axis=0)
    qd = jnp.take(q, dst, axis=0)
    vec = jnp.take(pos, dst, axis=0) - jnp.take(pos, src, axis=0)

    wkmat = (kW2.reshape(HID, DIN, DK) / (HID ** 0.5)).transpose(1, 0, 2).reshape(DIN, HID * DK)
    wvmat = (vW2.reshape(HID, DIN, DV) / (HID ** 0.5)).transpose(1, 0, 2).reshape(DIN, HID * DV)

    logits, v = pl.pallas_call(
        _edge_kernel,
        grid=(NSTEPS,),
        in_specs=[
            pl.BlockSpec((EB, 3), lambda i: (i, 0)),
            pl.BlockSpec((EB, DIN), lambda i: (i, 0)),
            pl.BlockSpec((EB, DQ), lambda i: (i, 0)),
            pl.BlockSpec((DIN, HID * DK), lambda i: (0, 0)),
            pl.BlockSpec((DIN, HID * DV), lambda i: (0, 0)),
            pl.BlockSpec((NB, HID), lambda i: (0, 0)),
            pl.BlockSpec((NB, HID), lambda i: (0, 0)),
            pl.BlockSpec((DQ, DK), lambda i: (0, 0)),
            pl.BlockSpec((DQ, DK), lambda i: (0, 0)),
            pl.BlockSpec((M1, 3 * M1), lambda i: (0, 0)),
            pl.BlockSpec((3, 3 * M1), lambda i: (0, 0)),
            pl.BlockSpec((M2, 5 * M2), lambda i: (0, 0)),
            pl.BlockSpec((5, 5 * M2), lambda i: (0, 0)),
        ],
        out_specs=[
            pl.BlockSpec((EB, 1), lambda i: (i, 0)),
            pl.BlockSpec((EB, DOUT), lambda i: (i, 0)),
        ],
        out_shape=[
            jax.ShapeDtypeStruct((E, 1), jnp.float32),
            jax.ShapeDtypeStruct((E, DOUT), jnp.float32),
        ],
    )(vec, xs, qd, wkmat, wvmat, kW1, vW1, W_tpq, W_sim,
      jnp.asarray(_R1), jnp.asarray(_T1), jnp.asarray(_R2), jnp.asarray(_T2))

    logits = logits[:, 0]
    mx = jax.ops.segment_max(logits, dst, num_segments=N)
    mx = jnp.where(jnp.isfinite(mx), mx, 0.0)
    ex = jnp.exp(logits - mx[dst])
    z = jax.ops.segment_sum(ex, dst, num_segments=N)
    alpha = ex / (z[dst] + 1e-9)
    out = jax.ops.segment_sum(alpha[:, None] * v, dst, num_segments=N)
    return out


# speedup vs baseline: 1.6392x; 1.6392x over previous
import numpy as np
import jax
import jax.numpy as jnp
from jax.experimental import pallas as pl
from jax.experimental.pallas import tpu as pltpu

N = 10000
E = 160000
DIN = 128
DQ = 32
DK = 16
NB = 32
HID = 16
M0, M1, M2 = 16, 8, 4
DV = M0 + M1 + M2
DOUT = M0 + 3 * M1 + 5 * M2

EB = 2000
NSTEPS = E // EB
QB = 2000

_R1 = np.kron(np.eye(M1), np.ones((1, 3))).astype(np.float32)
_T1 = np.kron(np.ones((1, M1)), np.eye(3)).astype(np.float32)
_R2 = np.kron(np.eye(M2), np.ones((1, 5))).astype(np.float32)
_T2 = np.kron(np.ones((1, M2)), np.eye(5)).astype(np.float32)


def _q_kernel(x_ref, wq_ref, q_ref):
    q_ref[...] = jnp.dot(x_ref[...], wq_ref[...],
                         preferred_element_type=jnp.float32) / (DIN ** 0.5)


def _edge_kernel(vec_ref, xs_ref, qd_ref, wkmat_ref, wvmat_ref, kw1_ref,
                 vw1_ref, wtpq_ref, wsim_ref, r1_ref, t1_ref, r2_ref, t2_ref,
                 logits_ref, v_ref, pmax_ref):
    vec = vec_ref[...]
    d2 = jnp.sum(vec * vec, axis=1, keepdims=True)
    d = jnp.sqrt(d2)
    u = vec / (d + 1e-9)
    sh1 = (3.0 ** 0.5) * u
    ux = u[:, 0:1]
    uy = u[:, 1:2]
    uz = u[:, 2:3]
    c2 = 15.0 ** 0.5
    sh2 = jnp.concatenate([
        c2 * ux * uy,
        c2 * uy * uz,
        (5.0 ** 0.5) * 0.5 * (3.0 * uz * uz - 1.0),
        c2 * ux * uz,
        0.5 * c2 * (ux * ux - uy * uy),
    ], axis=1)

    centers = jax.lax.broadcasted_iota(jnp.int32, (1, NB), 1).astype(jnp.float32) * (3.5 / (NB - 1))
    width = 3.5 / NB
    b = jnp.exp(-0.5 * ((d - centers) / width) ** 2)
    bn = b * (NB ** 0.5) / (jnp.sqrt(jnp.sum(b * b, axis=1, keepdims=True)) + 1e-9)

    hk = jax.nn.silu(jnp.dot(bn, kw1_ref[...],
                             preferred_element_type=jnp.float32) / (NB ** 0.5))
    hv = jax.nn.silu(jnp.dot(bn, vw1_ref[...],
                             preferred_element_type=jnp.float32) / (NB ** 0.5))

    xs = xs_ref[...]
    qd = qd_ref[...]

    tk = jnp.dot(xs, wkmat_ref[...], preferred_element_type=jnp.float32)
    k = jnp.zeros((EB, DK), dtype=jnp.float32)
    for h in range(HID):
        k = k + hk[:, h:h + 1] * tk[:, h * DK:(h + 1) * DK]
    k = k / (DIN ** 0.5) + jnp.dot(qd, wtpq_ref[...],
                                   preferred_element_type=jnp.float32) / (DQ ** 0.5)

    a = jnp.dot(qd, wsim_ref[...], preferred_element_type=jnp.float32)
    logits = jnp.sum(a * k, axis=1, keepdims=True) / ((DQ * DK) ** 0.5)

    tv = jnp.dot(xs, wvmat_ref[...], preferred_element_type=jnp.float32)
    s = jnp.zeros((EB, DV), dtype=jnp.float32)
    for h in range(HID):
        s = s + hv[:, h:h + 1] * tv[:, h * DV:(h + 1) * DV]
    s = s / (DIN ** 0.5)

    s0 = s[:, :M0]
    s1 = s[:, M0:M0 + M1]
    s2 = s[:, M0 + M1:]
    v1 = jnp.dot(s1, r1_ref[...], preferred_element_type=jnp.float32) * \
        jnp.dot(sh1, t1_ref[...], preferred_element_type=jnp.float32)
    v2 = jnp.dot(s2, r2_ref[...], preferred_element_type=jnp.float32) * \
        jnp.dot(sh2, t2_ref[...], preferred_element_type=jnp.float32)

    logits_ref[...] = logits
    v_ref[...] = jnp.concatenate([s0, v1, v2], axis=1)
    pmax_ref[...] = jnp.max(logits).reshape(1, 1, 1)


NACC = 4


def _scatter_kernel(dst_ref, gmax_ref, logits_ref, v_ref, out_ref,
                    rows_ref, acc_ref):
    i = pl.program_id(0)

    @pl.when(i == 0)
    def _init():
        acc_ref[...] = jnp.zeros_like(acc_ref)

    ex = jnp.exp(logits_ref[...] - gmax_ref[0, 0])
    v = v_ref[...]
    rows_ref[...] = jnp.concatenate(
        [ex, ex * v, jnp.zeros((EB, 3), jnp.float32)], axis=1)

    def body(j, carry):
        base = j * NACC
        for c in range(NACC):
            idx = dst_ref[0, 0, base + c]
            acc_ref[c, pl.ds(idx, 1), :] += rows_ref[pl.ds(base + c, 1), :]
        return carry

    jax.lax.fori_loop(0, EB // NACC, body, 0, unroll=2)

    @pl.when(i == NSTEPS - 1)
    def _fin():
        total = acc_ref[0] + acc_ref[1] + acc_ref[2] + acc_ref[3]
        z = total[:, 0:1]
        out_ref[...] = total / (z + 1e-9)


def kernel(x, pos, edge_index, W_query, W_tpq, W_sim, kW1, kW2, vW1, vW2):
    src = edge_index[0]
    dst = edge_index[1]

    q = pl.pallas_call(
        _q_kernel,
        grid=(N // QB,),
        in_specs=[
            pl.BlockSpec((QB, DIN), lambda i: (i, 0)),
            pl.BlockSpec((DIN, DQ), lambda i: (0, 0)),
        ],
        out_specs=pl.BlockSpec((QB, DQ), lambda i: (i, 0)),
        out_shape=jax.ShapeDtypeStruct((N, DQ), jnp.float32),
    )(x, W_query)

    xs = jnp.take(x, src, axis=0)
    qd = jnp.take(q, dst, axis=0)
    vec = jnp.take(pos, dst, axis=0) - jnp.take(pos, src, axis=0)

    wkmat = (kW2.reshape(HID, DIN, DK) / (HID ** 0.5)).transpose(1, 0, 2).reshape(DIN, HID * DK)
    wvmat = (vW2.reshape(HID, DIN, DV) / (HID ** 0.5)).transpose(1, 0, 2).reshape(DIN, HID * DV)

    logits, v, pmax = pl.pallas_call(
        _edge_kernel,
        grid=(NSTEPS,),
        in_specs=[
            pl.BlockSpec((EB, 3), lambda i: (i, 0)),
            pl.BlockSpec((EB, DIN), lambda i: (i, 0)),
            pl.BlockSpec((EB, DQ), lambda i: (i, 0)),
            pl.BlockSpec((DIN, HID * DK), lambda i: (0, 0)),
            pl.BlockSpec((DIN, HID * DV), lambda i: (0, 0)),
            pl.BlockSpec((NB, HID), lambda i: (0, 0)),
            pl.BlockSpec((NB, HID), lambda i: (0, 0)),
            pl.BlockSpec((DQ, DK), lambda i: (0, 0)),
            pl.BlockSpec((DQ, DK), lambda i: (0, 0)),
            pl.BlockSpec((M1, 3 * M1), lambda i: (0, 0)),
            pl.BlockSpec((3, 3 * M1), lambda i: (0, 0)),
            pl.BlockSpec((M2, 5 * M2), lambda i: (0, 0)),
            pl.BlockSpec((5, 5 * M2), lambda i: (0, 0)),
        ],
        out_specs=[
            pl.BlockSpec((EB, 1), lambda i: (i, 0)),
            pl.BlockSpec((EB, DOUT), lambda i: (i, 0)),
            pl.BlockSpec((1, 1, 1), lambda i: (i, 0, 0)),
        ],
        out_shape=[
            jax.ShapeDtypeStruct((E, 1), jnp.float32),
            jax.ShapeDtypeStruct((E, DOUT), jnp.float32),
            jax.ShapeDtypeStruct((NSTEPS, 1, 1), jnp.float32),
        ],
    )(vec, xs, qd, wkmat, wvmat, kW1, vW1, W_tpq, W_sim,
      jnp.asarray(_R1), jnp.asarray(_T1), jnp.asarray(_R2), jnp.asarray(_T2))

    gmax = jnp.max(pmax).reshape(1, 1)

    outk = pl.pallas_call(
        _scatter_kernel,
        grid=(NSTEPS,),
        in_specs=[
            pl.BlockSpec((1, 1, EB), lambda i: (i, 0, 0), memory_space=pltpu.SMEM),
            pl.BlockSpec((1, 1), lambda i: (0, 0), memory_space=pltpu.SMEM),
            pl.BlockSpec((EB, 1), lambda i: (i, 0)),
            pl.BlockSpec((EB, DOUT), lambda i: (i, 0)),
        ],
        out_specs=pl.BlockSpec((N, 64), lambda i: (0, 0)),
        out_shape=jax.ShapeDtypeStruct((N, 64), jnp.float32),
        scratch_shapes=[
            pltpu.VMEM((EB, 64), jnp.float32),
            pltpu.VMEM((NACC, N, 64), jnp.float32),
        ],
    )(dst.reshape(NSTEPS, 1, EB), gmax, logits, v)
    return outk[:, 1:DOUT + 1]
